# baseline (speedup 1.0000x reference)
import functools

import jax
import jax.numpy as jnp
from jax import lax
from jax.experimental import pallas as pl
from jax.experimental.pallas import tpu as pltpu
from jax.experimental.pallas import tpu_sc as plsc

BATCH = 16384
EMB = 32
NC, NS = 2, 16
NW = NC * NS
B_PER_W = BATCH // NW
CHUNK = 128
N_CHUNKS = B_PER_W // CHUNK


def _sc_gather(ut, tt, uid, tid):
    mesh = plsc.VectorSubcoreMesh(core_axis_name="c", subcore_axis_name="s")

    @functools.partial(
        pl.kernel,
        mesh=mesh,
        out_type=[
            jax.ShapeDtypeStruct((BATCH, EMB), jnp.float32),
            jax.ShapeDtypeStruct((BATCH, EMB), jnp.float32),
        ],
        scratch_types=[
            pltpu.VMEM((B_PER_W,), jnp.int32),
            pltpu.VMEM((B_PER_W,), jnp.int32),
            pltpu.VMEM((B_PER_W, EMB), jnp.float32),
            pltpu.VMEM((B_PER_W, EMB), jnp.float32),
            pltpu.SemaphoreType.DMA,
            pltpu.SemaphoreType.DMA,
        ],
        compiler_params=pltpu.CompilerParams(use_tc_tiling_on_sc=False),
    )
    def gather_kernel(ut_hbm, tt_hbm, uid_hbm, tid_hbm, ue_out, te_out,
                      uidx_v, tidx_v, urows_v, trows_v, usem, tsem):
        wid = lax.axis_index("s") * NC + lax.axis_index("c")
        base = wid * B_PER_W
        pltpu.sync_copy(uid_hbm.at[pl.ds(base, B_PER_W)], uidx_v)
        pltpu.sync_copy(tid_hbm.at[pl.ds(base, B_PER_W)], tidx_v)
        for c in range(N_CHUNKS):
            sl = pl.ds(c * CHUNK, CHUNK)
            pltpu.async_copy(ut_hbm.at[uidx_v.at[sl]], urows_v.at[sl], usem)
            pltpu.async_copy(tt_hbm.at[tidx_v.at[sl]], trows_v.at[sl], tsem)
        for c in range(N_CHUNKS):
            sl = pl.ds(c * CHUNK, CHUNK)
            pltpu.make_async_copy(
                ut_hbm.at[uidx_v.at[sl]], urows_v.at[sl], usem).wait()
            pltpu.make_async_copy(
                tt_hbm.at[tidx_v.at[sl]], trows_v.at[sl], tsem).wait()
        pltpu.sync_copy(urows_v, ue_out.at[pl.ds(base, B_PER_W), :])
        pltpu.sync_copy(trows_v, te_out.at[pl.ds(base, B_PER_W), :])

    return gather_kernel(ut, tt, uid, tid)


_BLK = 2048


def _mlp_body(ue, te, w1u, w1t, b1, w2, b2, w3, b3, out_ref):
    h1 = jnp.dot(ue[...], w1u[...], preferred_element_type=jnp.float32)
    h1 += jnp.dot(te[...], w1t[...], preferred_element_type=jnp.float32)
    h1 = jnp.maximum(h1 + b1[...], 0.0)
    h2 = jnp.dot(h1, w2[...], preferred_element_type=jnp.float32)
    h2 = jnp.maximum(h2 + b2[...], 0.0)
    out_ref[...] = jnp.dot(
        h2, w3[...], preferred_element_type=jnp.float32) + b3[...]


def _mlp(ue, te, w1u, w1t, b1, w2, b2, w3, b3):
    grid = (BATCH // _BLK,)
    whole = lambda i: (0, 0)
    return pl.pallas_call(
        _mlp_body,
        grid=grid,
        in_specs=[
            pl.BlockSpec((_BLK, EMB), lambda i: (i, 0)),
            pl.BlockSpec((_BLK, EMB), lambda i: (i, 0)),
            pl.BlockSpec((EMB, 256), whole),
            pl.BlockSpec((EMB, 256), whole),
            pl.BlockSpec((1, 256), whole),
            pl.BlockSpec((256, 64), whole),
            pl.BlockSpec((1, 64), whole),
            pl.BlockSpec((64, 1), whole),
            pl.BlockSpec((1, 1), whole),
        ],
        out_specs=pl.BlockSpec((_BLK, 1), lambda i: (i, 0)),
        out_shape=jax.ShapeDtypeStruct((BATCH, 1), jnp.float32),
    )(ue, te, w1u, w1t, b1, w2, b2, w3, b3)


def kernel(user_id, task_id, user_table, task_table, W1, b1, W2, b2, W3, b3):
    uid = user_id.astype(jnp.int32)
    tid = task_id.astype(jnp.int32)
    ue, te = _sc_gather(user_table, task_table, uid, tid)
    return _mlp(ue, te, W1[:EMB], W1[EMB:], b1.reshape(1, 256),
                W2, b2.reshape(1, 64), W3, b3.reshape(1, 1))


# speedup vs baseline: 1.3212x; 1.3212x over previous
import functools

import jax
import jax.numpy as jnp
from jax import lax
from jax.experimental import pallas as pl
from jax.experimental.pallas import tpu as pltpu
from jax.experimental.pallas import tpu_sc as plsc

BATCH = 16384
EMB = 32
NC, NS = 2, 16
NW = NC * NS
B_PER_W = BATCH // NW
CHUNK = 32
N_CHUNKS = B_PER_W // CHUNK
BLK_ROWS = 8


def _sc_gather(ut, tt, uid, tid):
    mesh = plsc.VectorSubcoreMesh(core_axis_name="c", subcore_axis_name="s")

    @functools.partial(
        pl.kernel,
        mesh=mesh,
        out_type=[
            jax.ShapeDtypeStruct((BATCH, EMB), jnp.float32),
            jax.ShapeDtypeStruct((BATCH, EMB), jnp.float32),
        ],
        scratch_types=[
            pltpu.VMEM((B_PER_W,), jnp.int32),
            pltpu.VMEM((B_PER_W,), jnp.int32),
            pltpu.VMEM((CHUNK * BLK_ROWS, EMB), jnp.float32),
            pltpu.VMEM((CHUNK * BLK_ROWS, EMB), jnp.float32),
            pltpu.VMEM((CHUNK, EMB), jnp.float32),
            pltpu.VMEM((CHUNK, EMB), jnp.float32),
            pltpu.SemaphoreType.DMA,
            pltpu.SemaphoreType.DMA,
        ],
        compiler_params=pltpu.CompilerParams(needs_layout_passes=False),
    )
    def gather_kernel(ut_hbm, tt_hbm, uid_hbm, tid_hbm, ue_out, te_out,
                      uidx_v, tidx_v, ublk_v, tblk_v, ucmp_v, tcmp_v,
                      usem, tsem):
        wid = lax.axis_index("s") * NC + lax.axis_index("c")
        base = wid * B_PER_W
        pltpu.sync_copy(uid_hbm.at[pl.ds(base, B_PER_W)], uidx_v)
        pltpu.sync_copy(tid_hbm.at[pl.ds(base, B_PER_W)], tidx_v)

        lanes_lo = lax.iota(jnp.int32, 16)
        lanes_hi = lanes_lo + 16

        def chunk(c, _):
            cbase = c * CHUNK
            ius = [uidx_v[pl.ds(cbase + g * 16, 16)]
                   for g in range(CHUNK // 16)]
            its = [tidx_v[pl.ds(cbase + g * 16, 16)]
                   for g in range(CHUNK // 16)]
            for k in range(CHUNK):
                iu_k = ius[k // 16][k % 16]
                it_k = its[k // 16][k % 16]
                ub = pl.multiple_of((iu_k // BLK_ROWS) * BLK_ROWS, BLK_ROWS)
                tb = pl.multiple_of((it_k // BLK_ROWS) * BLK_ROWS, BLK_ROWS)
                pltpu.async_copy(
                    ut_hbm.at[pl.ds(ub, BLK_ROWS), :],
                    ublk_v.at[pl.ds(k * BLK_ROWS, BLK_ROWS), :], usem)
                pltpu.async_copy(
                    tt_hbm.at[pl.ds(tb, BLK_ROWS), :],
                    tblk_v.at[pl.ds(k * BLK_ROWS, BLK_ROWS), :], tsem)
            for k in range(CHUNK):
                pltpu.make_async_copy(
                    ut_hbm.at[pl.ds(0, BLK_ROWS), :],
                    ublk_v.at[pl.ds(k * BLK_ROWS, BLK_ROWS), :], usem).wait()
                pltpu.make_async_copy(
                    tt_hbm.at[pl.ds(0, BLK_ROWS), :],
                    tblk_v.at[pl.ds(k * BLK_ROWS, BLK_ROWS), :], tsem).wait()
            for k in range(CHUNK):
                ur = jnp.full((16,), k * BLK_ROWS, jnp.int32) + (
                    ius[k // 16][k % 16] % BLK_ROWS)
                tr = jnp.full((16,), k * BLK_ROWS, jnp.int32) + (
                    its[k // 16][k % 16] % BLK_ROWS)
                ucmp_v[k, pl.ds(0, 16)] = plsc.load_gather(
                    ublk_v, [ur, lanes_lo])
                ucmp_v[k, pl.ds(16, 16)] = plsc.load_gather(
                    ublk_v, [ur, lanes_hi])
                tcmp_v[k, pl.ds(0, 16)] = plsc.load_gather(
                    tblk_v, [tr, lanes_lo])
                tcmp_v[k, pl.ds(16, 16)] = plsc.load_gather(
                    tblk_v, [tr, lanes_hi])
            pltpu.sync_copy(ucmp_v, ue_out.at[pl.ds(base + cbase, CHUNK), :])
            pltpu.sync_copy(tcmp_v, te_out.at[pl.ds(base + cbase, CHUNK), :])
            return ()

        lax.fori_loop(0, N_CHUNKS, chunk, (), unroll=False)

    return gather_kernel(ut, tt, uid, tid)


_BLK = 2048


def _mlp_body(ue, te, w1u, w1t, b1, w2, b2, w3, b3, out_ref):
    h1 = jnp.dot(ue[...], w1u[...], preferred_element_type=jnp.float32)
    h1 += jnp.dot(te[...], w1t[...], preferred_element_type=jnp.float32)
    h1 = jnp.maximum(h1 + b1[...], 0.0)
    h2 = jnp.dot(h1, w2[...], preferred_element_type=jnp.float32)
    h2 = jnp.maximum(h2 + b2[...], 0.0)
    out_ref[...] = jnp.dot(
        h2, w3[...], preferred_element_type=jnp.float32) + b3[...]


def _mlp(ue, te, w1u, w1t, b1, w2, b2, w3, b3):
    grid = (BATCH // _BLK,)
    whole = lambda i: (0, 0)
    return pl.pallas_call(
        _mlp_body,
        grid=grid,
        in_specs=[
            pl.BlockSpec((_BLK, EMB), lambda i: (i, 0)),
            pl.BlockSpec((_BLK, EMB), lambda i: (i, 0)),
            pl.BlockSpec((EMB, 256), whole),
            pl.BlockSpec((EMB, 256), whole),
            pl.BlockSpec((1, 256), whole),
            pl.BlockSpec((256, 64), whole),
            pl.BlockSpec((1, 64), whole),
            pl.BlockSpec((64, 1), whole),
            pl.BlockSpec((1, 1), whole),
        ],
        out_specs=pl.BlockSpec((_BLK, 1), lambda i: (i, 0)),
        out_shape=jax.ShapeDtypeStruct((BATCH, 1), jnp.float32),
    )(ue, te, w1u, w1t, b1, w2, b2, w3, b3)


def kernel(user_id, task_id, user_table, task_table, W1, b1, W2, b2, W3, b3):
    uid = user_id.astype(jnp.int32)
    tid = task_id.astype(jnp.int32)
    ue, te = _sc_gather(user_table, task_table, uid, tid)
    return _mlp(ue, te, W1[:EMB], W1[EMB:], b1.reshape(1, 256),
                W2, b2.reshape(1, 64), W3, b3.reshape(1, 1))
